# baseline (speedup 1.0000x reference)
import jax
import jax.numpy as jnp
from jax.experimental import pallas as pl
from jax.experimental.pallas import tpu as pltpu


def _mm_body(x_ref, p_ref, o_ref, pbf_ref):
    @pl.when(pl.program_id(0) == 0)
    def _cache_p():
        pbf_ref[...] = p_ref[...].astype(jnp.bfloat16)

    xb = x_ref[...].astype(jnp.bfloat16)
    o_ref[...] = jax.lax.dot_general(
        xb, pbf_ref[...], (((1,), (1,)), ((), ())),
        preferred_element_type=jnp.float32)


def kernel(x, P):
    orig_shape = (*x.shape[:-1], P.shape[0])
    x2 = x.reshape(-1, x.shape[-1])
    M, K = x2.shape
    N = P.shape[0]
    BM = 256
    out = pl.pallas_call(
        _mm_body,
        grid=(M // BM,),
        in_specs=[
            pl.BlockSpec((BM, K), lambda i: (i, 0)),
            pl.BlockSpec((N, K), lambda i: (0, 0)),
        ],
        out_specs=pl.BlockSpec((BM, N), lambda i: (i, 0)),
        out_shape=jax.ShapeDtypeStruct((M, N), jnp.float32),
        scratch_shapes=[pltpu.VMEM((N, K), jnp.bfloat16)],
        compiler_params=pltpu.CompilerParams(
            vmem_limit_bytes=63 * 1024 * 1024),
    )(x2, P)
    return out.reshape(orig_shape)


# speedup vs baseline: 1.0253x; 1.0253x over previous
import jax
import jax.numpy as jnp
from jax.experimental import pallas as pl
from jax.experimental.pallas import tpu as pltpu


def _mm_body(x_ref, p_ref, o_ref):
    xb = x_ref[...].astype(jnp.bfloat16)
    pb = p_ref[...].astype(jnp.bfloat16)
    o_ref[...] = jax.lax.dot_general(
        xb, pb, (((1,), (1,)), ((), ())),
        preferred_element_type=jnp.float32)


def kernel(x, P):
    orig_shape = (*x.shape[:-1], P.shape[0])
    x2 = x.reshape(-1, x.shape[-1])
    M, K = x2.shape
    N = P.shape[0]
    BM = 512
    out = pl.pallas_call(
        _mm_body,
        grid=(M // BM,),
        in_specs=[
            pl.BlockSpec((BM, K), lambda i: (i, 0)),
            pl.BlockSpec((N, K), lambda i: (0, 0)),
        ],
        out_specs=pl.BlockSpec((BM, N), lambda i: (i, 0)),
        out_shape=jax.ShapeDtypeStruct((M, N), jnp.float32),
        compiler_params=pltpu.CompilerParams(
            vmem_limit_bytes=63 * 1024 * 1024),
    )(x2, P)
    return out.reshape(orig_shape)
